# baseline (speedup 1.0000x reference)
import jax
import jax.numpy as jnp
from jax.experimental import pallas as pl

_B = 1024
_H = 50
_K = 64
_CHUNK = 256
_C0 = _H * _K


def _probe_body(x_ref, seen_ref, g_ref, xout_ref, out_ref):
    i = pl.program_id(0)
    xb = x_ref[...]
    xout_ref[...] = xb
    a = jnp.abs(xb[:, :_C0])
    s = jax.lax.dot_general(
        a, g_ref[...], (((1,), (0,)), ((), ())),
        preferred_element_type=jnp.float32,
    )
    rz = (s == 0.0).astype(jnp.float32)

    @pl.when(i < _B // _CHUNK)
    def _init():
        out_ref[...] = seen_ref[...] + rz

    @pl.when(i >= _B // _CHUNK)
    def _acc():
        out_ref[...] += rz


def kernel(x, seen):
    n = x.shape[0]
    row = 2 * _C0
    grid = n // _CHUNK
    blocks_per_b = _B // _CHUNK
    x_flat = x.reshape(n, row)
    g = (jnp.arange(_C0)[:, None] // _K == jnp.arange(_H)[None, :]).astype(
        jnp.float32
    )
    x_out, seen_new = pl.pallas_call(
        _probe_body,
        grid=(grid,),
        in_specs=[
            pl.BlockSpec((_CHUNK, row), lambda i: (i, 0)),
            pl.BlockSpec((_CHUNK, _H), lambda i: (i % blocks_per_b, 0)),
            pl.BlockSpec((_C0, _H), lambda i: (0, 0)),
        ],
        out_specs=[
            pl.BlockSpec((_CHUNK, row), lambda i: (i, 0)),
            pl.BlockSpec((_CHUNK, _H), lambda i: (i % blocks_per_b, 0)),
        ],
        out_shape=[
            jax.ShapeDtypeStruct((n, row), jnp.float32),
            jax.ShapeDtypeStruct((_B, _H), jnp.float32),
        ],
    )(x_flat, seen, g)
    return (x_out.reshape(x.shape), seen_new)
